# baseline (speedup 1.0000x reference)
import jax
import jax.numpy as jnp
from jax import lax
from jax.experimental import pallas as pl
from jax.experimental.pallas import tpu as pltpu
from jax.experimental.pallas import tpu_sc as plsc

N_ROWS = 1_000_000
DIM = 32
EPS2 = 1e-16

NUM_CORES = 2
NUM_SUBCORES = 16
NUM_WORKERS = NUM_CORES * NUM_SUBCORES
CHUNK = 1000
NUM_CHUNKS = N_ROWS // CHUNK
BIG_WORKERS = NUM_CHUNKS - 31 * NUM_WORKERS
MAX_CHUNKS_PER_WORKER = 32
GROUPS = (CHUNK + 15) // 16


def _rsqrt_newton(ss):
    i = plsc.bitcast(ss, jnp.int32)
    i = jnp.int32(0x5F3759DF) - lax.shift_right_logical(i, 1)
    r = plsc.bitcast(i, jnp.float32)
    for _ in range(3):
        r = r * (1.5 - 0.5 * ss * r * r)
    return r


def _compute_chunk(in_buf, out_buf, lane_iota):
    @pl.loop(0, GROUPS)
    def _group(g):
        base = jnp.minimum(g * 16, CHUNK - 16)
        rows = base + lane_iota
        ss = jnp.zeros((16,), jnp.float32)
        xs = []
        for f in range(DIM):
            col = jnp.full((16,), f, jnp.int32)
            x = plsc.load_gather(in_buf, [rows, col])
            xs.append(x)
            ss = ss + x * x
        ss = jnp.maximum(ss, EPS2)
        r = _rsqrt_newton(ss)
        n = ss * r
        scale = (jnp.exp(n) - jnp.exp(-n)) * 0.5 * r
        scale = jnp.where(n < 1e-3, 1.0 + ss * (1.0 / 6.0), scale)
        for f in range(DIM):
            col = jnp.full((16,), f, jnp.int32)
            plsc.store_scatter(out_buf, [rows, col], xs[f] * scale)


def _body(x_hbm, out_hbm, in_bufs, out_bufs, load_sems, store_sems):
    wid = lax.axis_index("s") * NUM_CORES + lax.axis_index("c")
    lane_iota = lax.iota(jnp.int32, 16)

    is_big = wid < BIG_WORKERS
    base_chunk = jnp.where(is_big, 32 * wid, 31 * wid + BIG_WORKERS)

    def start_load(k, b):
        pltpu.async_copy(
            x_hbm.at[pl.ds((base_chunk + k) * CHUNK, CHUNK)],
            in_bufs[b],
            load_sems[b],
        )

    def wait_load(b):
        pltpu.make_async_copy(
            x_hbm.at[pl.ds(0, CHUNK)], in_bufs[b], load_sems[b]
        ).wait()

    def start_store(k, b):
        pltpu.async_copy(
            out_bufs[b],
            out_hbm.at[pl.ds((base_chunk + k) * CHUNK, CHUNK)],
            store_sems[b],
        )

    def wait_store(b):
        pltpu.make_async_copy(
            x_hbm.at[pl.ds(0, CHUNK)], out_bufs[b], store_sems[b]
        ).wait()

    start_load(0, 0)
    start_load(1, 1)

    @pl.loop(0, 15)
    def _pair(p):
        for b in range(2):
            k = 2 * p + b
            wait_load(b)

            @pl.when(p >= 1)
            def _():
                wait_store(b)

            _compute_chunk(in_bufs[b], out_bufs[b], lane_iota)
            start_store(k, b)
            if b == 0:
                start_load(k + 2, b)
            else:

                @pl.when((p < 14) | is_big)
                def _():
                    start_load(k + 2, b)

    wait_load(0)
    wait_store(0)
    _compute_chunk(in_bufs[0], out_bufs[0], lane_iota)
    start_store(30, 0)

    @pl.when(is_big)
    def _():
        wait_load(1)
        wait_store(1)
        _compute_chunk(in_bufs[1], out_bufs[1], lane_iota)
        start_store(31, 1)

    wait_store(0)
    wait_store(1)


@jax.jit
def kernel(tangent_embeddings):
    mesh = plsc.VectorSubcoreMesh(
        core_axis_name="c",
        subcore_axis_name="s",
        num_cores=NUM_CORES,
        num_subcores=NUM_SUBCORES,
    )
    f = pl.kernel(
        _body,
        out_type=jax.ShapeDtypeStruct((N_ROWS, DIM), jnp.float32),
        mesh=mesh,
        scratch_types=dict(
            in_bufs=[pltpu.VMEM((CHUNK, DIM), jnp.float32) for _ in range(2)],
            out_bufs=[pltpu.VMEM((CHUNK, DIM), jnp.float32) for _ in range(2)],
            load_sems=[pltpu.SemaphoreType.DMA for _ in range(2)],
            store_sems=[pltpu.SemaphoreType.DMA for _ in range(2)],
        ),
        compiler_params=pltpu.CompilerParams(
            needs_layout_passes=False, use_tc_tiling_on_sc=False
        ),
        name="lorentz_exp_map0_sc",
    )
    return f(tangent_embeddings)


# speedup vs baseline: 2.0468x; 2.0468x over previous
import jax
import jax.numpy as jnp
from jax import lax
from jax.experimental import pallas as pl
from jax.experimental.pallas import tpu as pltpu
from jax.experimental.pallas import tpu_sc as plsc

N_ROWS = 1_000_000
DIM = 32
EPS2 = 1e-16

NUM_CORES = 2
NUM_SUBCORES = 16
NUM_WORKERS = NUM_CORES * NUM_SUBCORES
CHUNK = 248
MAIN_CHUNKS = N_ROWS // CHUNK
LAST_ROW0 = N_ROWS - CHUNK
COMMON = MAIN_CHUNKS // NUM_WORKERS
PAIRS = COMMON // 2
GROUPS = 16
LAST_GROUP_BASE = CHUNK - 16

def _rsqrt_newton(ss):
    i = plsc.bitcast(ss, jnp.int32)
    i = jnp.int32(0x5F3759DF) - lax.shift_right_logical(i, 1)
    r = plsc.bitcast(i, jnp.float32)
    for _ in range(3):
        r = r * (1.5 - 0.5 * ss * r * r)
    return r


def _compute_chunk(in_buf, out_buf, stage, lane_iota):
    iota17 = lane_iota * 17

    @pl.loop(0, GROUPS)
    def _group(g):
        base = pl.multiple_of(jnp.minimum(g * 16, LAST_GROUP_BASE), 8)
        lo = []
        hi = []
        for j in range(16):
            a = in_buf[base + j, pl.ds(0, 16)]
            b = in_buf[base + j, pl.ds(16, 16)]
            lo.append(a)
            hi.append(b)
            stage[pl.ds(17 * j, 16)] = a * a + b * b
        ss = jnp.zeros((16,), jnp.float32)
        for c in range(16):
            ss = ss + plsc.load_gather(stage, [iota17 + c])
        ss = jnp.maximum(ss, EPS2)
        r = _rsqrt_newton(ss)
        n = ss * r
        scale = (jnp.exp(n) - jnp.exp(-n)) * 0.5 * r
        scale = jnp.where(n < 1e-3, 1.0 + ss * (1.0 / 6.0), scale)
        for j in range(16):
            s_j = jnp.full((16,), scale[j], jnp.float32)
            out_buf[base + j, pl.ds(0, 16)] = lo[j] * s_j
            out_buf[base + j, pl.ds(16, 16)] = hi[j] * s_j


def _body(
    x_hbm,
    out_hbm,
    in_bufs,
    out_bufs,
    stage,
    load_sems,
    store_sems,
):
    wid = lax.axis_index("s") * NUM_CORES + lax.axis_index("c")
    lane_iota = lax.iota(jnp.int32, 16)

    base_chunk = jnp.where(wid == 0, 0, COMMON * wid + 1)

    def row0_of(k):
        return pl.multiple_of(
            jnp.minimum((base_chunk + k) * CHUNK, LAST_ROW0), 8
        )

    def start_load(k, b):
        pltpu.async_copy(
            x_hbm.at[pl.ds(row0_of(k), CHUNK)], in_bufs[b], load_sems[b]
        )

    def wait_load(b):
        pltpu.make_async_copy(
            x_hbm.at[pl.ds(0, CHUNK)], in_bufs[b], load_sems[b]
        ).wait()

    def start_store(k, b):
        pltpu.async_copy(
            out_bufs[b], out_hbm.at[pl.ds(row0_of(k), CHUNK)], store_sems[b]
        )

    def wait_store(b):
        pltpu.make_async_copy(
            x_hbm.at[pl.ds(0, CHUNK)], out_bufs[b], store_sems[b]
        ).wait()

    start_load(0, 0)
    start_load(1, 1)

    @pl.loop(0, PAIRS)
    def _pair(p):
        for b in range(2):
            k = 2 * p + b
            wait_load(b)

            @pl.when(p >= 1)
            def _():
                wait_store(b)

            _compute_chunk(in_bufs[b], out_bufs[b], stage, lane_iota)
            start_store(k, b)
            if b == 0:

                @pl.when((p < PAIRS - 1) | (wid == 0))
                def _():
                    start_load(k + 2, b)

            else:

                @pl.when(p < PAIRS - 1)
                def _():
                    start_load(k + 2, b)

    @pl.when(wid == 0)
    def _():
        wait_load(0)
        wait_store(0)
        _compute_chunk(in_bufs[0], out_bufs[0], stage, lane_iota)
        start_store(COMMON, 0)

    wait_store(0)
    wait_store(1)


@jax.jit
def kernel(tangent_embeddings):
    mesh = plsc.VectorSubcoreMesh(
        core_axis_name="c",
        subcore_axis_name="s",
        num_cores=NUM_CORES,
        num_subcores=NUM_SUBCORES,
    )
    f = pl.kernel(
        _body,
        out_type=jax.ShapeDtypeStruct((N_ROWS, DIM), jnp.float32),
        mesh=mesh,
        scratch_types=dict(
            in_bufs=[pltpu.VMEM((CHUNK, DIM), jnp.float32) for _ in range(2)],
            out_bufs=[pltpu.VMEM((CHUNK, DIM), jnp.float32) for _ in range(2)],
            stage=pltpu.VMEM((16 * 17,), jnp.float32),
            load_sems=[pltpu.SemaphoreType.DMA for _ in range(2)],
            store_sems=[pltpu.SemaphoreType.DMA for _ in range(2)],
        ),
        compiler_params=pltpu.CompilerParams(
            needs_layout_passes=False, use_tc_tiling_on_sc=True
        ),
        name="lorentz_exp_map0_sc",
    )
    return f(tangent_embeddings)
